# Initial kernel scaffold; baseline (speedup 1.0000x reference)
import jax
import jax.numpy as jnp
from jax.experimental import pallas as pl


def kernel(anchors, pred_deltas, scores):
    raise NotImplementedError("write your pallas kernel here")


# speedup vs baseline: 22.3684x; 22.3684x over previous
import jax
import jax.numpy as jnp
from jax import lax
from jax.experimental import pallas as pl
from jax.experimental.pallas import tpu as pltpu

_N = 20000
_PAD = 20480
_R = 160
_C = 128
_PRE = 2000
_POST = 1000
_TH = 0.7
_NEG = -1e30
_IMG_H = 1024.0
_IMG_W = 1024.0
_INT_MIN = -2147483648
_INT_MAX = 2147483647


def _nms_body(a0, a1, a2, a3, d0, d1, d2, d3, s,
              ox1, oy1, ox2, oy2, osc,
              x1r, y1r, x2r, y2r, arear, curr):
    A0, A1, A2, A3 = a0[...], a1[...], a2[...], a3[...]
    D0, D1, D2, D3 = d0[...], d1[...], d2[...], d3[...]
    S = s[...]

    ctr_x = (A0 + A2) / 2.0
    ctr_y = (A1 + A3) / 2.0
    w = A2 - A0
    h = A3 - A1
    x1 = jnp.minimum(jnp.maximum(ctr_x - D0 * w, 0.0), _IMG_W)
    y1 = jnp.minimum(jnp.maximum(ctr_y - D1 * h, 0.0), _IMG_H)
    x2 = jnp.minimum(jnp.maximum(ctr_x + D2 * w, 0.0), _IMG_W)
    y2 = jnp.minimum(jnp.maximum(ctr_y + D3 * h, 0.0), _IMG_H)

    row = lax.broadcasted_iota(jnp.int32, (_R, _C), 0)
    col = lax.broadcasted_iota(jnp.int32, (_R, _C), 1)
    idx = row * _C + col
    inb = idx < _N

    valid = (x2 - x1 > 0.0) & (y2 - y1 > 0.0) & inb
    ms = jnp.where(valid, S, _NEG)

    b = lax.bitcast_convert_type(ms, jnp.int32)
    key = jnp.where(b >= 0, b, _INT_MIN - b)

    def bs1(_, lohi):
        lo, hi = lohi
        mid = (lo >> 1) + (hi >> 1) + (lo & hi & 1)
        cnt = jnp.sum((inb & (key > mid)).astype(jnp.int32))
        go_hi = cnt < _PRE
        live = lo < hi
        new_lo = jnp.where(live & ~go_hi, mid + 1, lo)
        new_hi = jnp.where(live & go_hi, mid, hi)
        return new_lo, new_hi

    tau, _unused_hi = lax.fori_loop(
        0, 32, bs1, (jnp.int32(_INT_MIN), jnp.int32(_INT_MAX)))

    n_gt = jnp.sum((inb & (key > tau)).astype(jnp.int32))
    quota = _PRE - n_gt
    tie = inb & (key == tau)

    def bs2(_, lohi):
        lo, hi = lohi
        mid = (lo + hi) // 2
        cnt = jnp.sum((tie & (idx < mid)).astype(jnp.int32))
        go_hi = cnt >= quota
        live = lo < hi
        new_lo = jnp.where(live & ~go_hi, mid + 1, lo)
        new_hi = jnp.where(live & go_hi, mid, hi)
        return new_lo, new_hi

    mcut, _unused_hi2 = lax.fori_loop(0, 15, bs2, (jnp.int32(0), jnp.int32(_PAD)))

    pool = inb & ((key > tau) | (tie & (idx < mcut)))

    x1r[...] = x1
    y1r[...] = y1
    x2r[...] = x2
    y2r[...] = y2
    arear[...] = (x2 - x1) * (y2 - y1)
    curr[...] = jnp.where(pool, ms, _NEG)

    zero8 = jnp.zeros((8, _C), jnp.float32)
    ox1[...] = zero8
    oy1[...] = zero8
    ox2[...] = zero8
    oy2[...] = zero8
    osc[...] = zero8

    orow = lax.broadcasted_iota(jnp.int32, (8, _C), 0)
    ocol = lax.broadcasted_iota(jnp.int32, (8, _C), 1)
    oidx = orow * _C + ocol

    def step(t, carry):
        cur = curr[...]
        m = jnp.max(cur)
        j = jnp.min(jnp.where(cur == m, idx, _PAD))
        selj = idx == j

        X1 = x1r[...]
        Y1 = y1r[...]
        X2 = x2r[...]
        Y2 = y2r[...]
        AR = arear[...]

        bx1 = jnp.max(jnp.where(selj, X1, _NEG))
        by1 = jnp.max(jnp.where(selj, Y1, _NEG))
        bx2 = jnp.max(jnp.where(selj, X2, _NEG))
        by2 = jnp.max(jnp.where(selj, Y2, _NEG))

        ix1 = jnp.maximum(bx1, X1)
        iy1 = jnp.maximum(by1, Y1)
        ix2 = jnp.minimum(bx2, X2)
        iy2 = jnp.minimum(by2, Y2)
        inter = jnp.maximum(ix2 - ix1, 0.0) * jnp.maximum(iy2 - iy1, 0.0)
        area1 = (bx2 - bx1) * (by2 - by1)
        union = area1 + AR - inter
        iou = inter / jnp.maximum(union, 1e-6)
        supp = (iou >= _TH) | selj
        curr[...] = jnp.where(supp, _NEG, cur)

        ok = m > (_NEG / 2.0)
        hit = oidx == t
        ox1[...] = jnp.where(hit, jnp.where(ok, bx1, 0.0), ox1[...])
        oy1[...] = jnp.where(hit, jnp.where(ok, by1, 0.0), oy1[...])
        ox2[...] = jnp.where(hit, jnp.where(ok, bx2, 0.0), ox2[...])
        oy2[...] = jnp.where(hit, jnp.where(ok, by2, 0.0), oy2[...])
        osc[...] = jnp.where(hit, jnp.where(ok, m, 0.0), osc[...])
        return carry

    lax.fori_loop(0, _POST, step, 0)


def _call(interpret=False):
    return pl.pallas_call(
        _nms_body,
        out_shape=[jax.ShapeDtypeStruct((8, _C), jnp.float32)] * 5,
        scratch_shapes=[pltpu.VMEM((_R, _C), jnp.float32)] * 6,
        interpret=interpret,
    )


def kernel(anchors, pred_deltas, scores):
    pad = _PAD - _N

    def col(x, i):
        return jnp.pad(x[:, i], (0, pad)).reshape(_R, _C)

    a = [col(anchors, i) for i in range(4)]
    d = [col(pred_deltas, i) for i in range(4)]
    s = jnp.pad(scores, (0, pad)).reshape(_R, _C)
    outs = _call()(*a, *d, s)
    cols = [o.reshape(-1)[:_POST] for o in outs]
    return jnp.stack(cols, axis=1)


# speedup vs baseline: 23.4004x; 1.0461x over previous
import jax
import jax.numpy as jnp
from jax import lax
from jax.experimental import pallas as pl
from jax.experimental.pallas import tpu as pltpu

_N = 20000
_PAD = 20480
_R = 160
_C = 128
_PRE = 2000
_POST = 1000
_TH = 0.7
_NEG = -1e30
_IMG_H = 1024.0
_IMG_W = 1024.0
_INT_MIN = -2147483648
_INT_MAX = 2147483647


def _nms_body(a0, a1, a2, a3, d0, d1, d2, d3, s,
              ox1, oy1, ox2, oy2, osc,
              x1r, y1r, x2r, y2r, arear, curr):
    A0, A1, A2, A3 = a0[...], a1[...], a2[...], a3[...]
    D0, D1, D2, D3 = d0[...], d1[...], d2[...], d3[...]
    S = s[...]

    ctr_x = (A0 + A2) / 2.0
    ctr_y = (A1 + A3) / 2.0
    w = A2 - A0
    h = A3 - A1
    x1 = jnp.minimum(jnp.maximum(ctr_x - D0 * w, 0.0), _IMG_W)
    y1 = jnp.minimum(jnp.maximum(ctr_y - D1 * h, 0.0), _IMG_H)
    x2 = jnp.minimum(jnp.maximum(ctr_x + D2 * w, 0.0), _IMG_W)
    y2 = jnp.minimum(jnp.maximum(ctr_y + D3 * h, 0.0), _IMG_H)

    row = lax.broadcasted_iota(jnp.int32, (_R, _C), 0)
    col = lax.broadcasted_iota(jnp.int32, (_R, _C), 1)
    idx = row * _C + col
    inb = idx < _N

    valid = (x2 - x1 > 0.0) & (y2 - y1 > 0.0) & inb
    ms = jnp.where(valid, S, _NEG)

    b = lax.bitcast_convert_type(ms, jnp.int32)
    key = jnp.where(b >= 0, b, _INT_MIN - b)

    def bs1(_, lohi):
        lo, hi = lohi
        mid = (lo >> 1) + (hi >> 1) + (lo & hi & 1)
        cnt = jnp.sum((inb & (key > mid)).astype(jnp.int32))
        go_hi = cnt < _PRE
        live = lo < hi
        new_lo = jnp.where(live & ~go_hi, mid + 1, lo)
        new_hi = jnp.where(live & go_hi, mid, hi)
        return new_lo, new_hi

    tau, _unused_hi = lax.fori_loop(
        0, 32, bs1, (jnp.int32(_INT_MIN), jnp.int32(_INT_MAX)))

    n_gt = jnp.sum((inb & (key > tau)).astype(jnp.int32))
    quota = _PRE - n_gt
    tie = inb & (key == tau)

    def bs2(_, lohi):
        lo, hi = lohi
        mid = (lo + hi) // 2
        cnt = jnp.sum((tie & (idx < mid)).astype(jnp.int32))
        go_hi = cnt >= quota
        live = lo < hi
        new_lo = jnp.where(live & ~go_hi, mid + 1, lo)
        new_hi = jnp.where(live & go_hi, mid, hi)
        return new_lo, new_hi

    mcut, _unused_hi2 = lax.fori_loop(0, 15, bs2, (jnp.int32(0), jnp.int32(_PAD)))

    pool = inb & ((key > tau) | (tie & (idx < mcut)))

    x1r[...] = x1
    y1r[...] = y1
    x2r[...] = x2
    y2r[...] = y2
    arear[...] = (x2 - x1) * (y2 - y1)
    curr[...] = jnp.where(pool, ms, _NEG)

    zero8 = jnp.zeros((8, _C), jnp.float32)
    ox1[...] = zero8
    oy1[...] = zero8
    ox2[...] = zero8
    oy2[...] = zero8
    osc[...] = zero8

    orow = lax.broadcasted_iota(jnp.int32, (8, _C), 0)
    ocol = lax.broadcasted_iota(jnp.int32, (8, _C), 1)
    oidx = orow * _C + ocol

    lane = lax.broadcasted_iota(jnp.int32, (1, _C), 1)

    def step(t, carry):
        cur = curr[...]
        m = jnp.max(cur)
        j = jnp.min(jnp.where(cur == m, idx, _PAD))
        selj = idx == j

        X1 = x1r[...]
        Y1 = y1r[...]
        X2 = x2r[...]
        Y2 = y2r[...]
        AR = arear[...]

        r = j >> 7
        c = j & 127
        lm = lane == c
        bx1 = jnp.max(jnp.where(lm, x1r[pl.ds(r, 1), :], _NEG))
        by1 = jnp.max(jnp.where(lm, y1r[pl.ds(r, 1), :], _NEG))
        bx2 = jnp.max(jnp.where(lm, x2r[pl.ds(r, 1), :], _NEG))
        by2 = jnp.max(jnp.where(lm, y2r[pl.ds(r, 1), :], _NEG))

        ix1 = jnp.maximum(bx1, X1)
        iy1 = jnp.maximum(by1, Y1)
        ix2 = jnp.minimum(bx2, X2)
        iy2 = jnp.minimum(by2, Y2)
        inter = jnp.maximum(ix2 - ix1, 0.0) * jnp.maximum(iy2 - iy1, 0.0)
        area1 = (bx2 - bx1) * (by2 - by1)
        union = area1 + AR - inter
        iou = inter / jnp.maximum(union, 1e-6)
        supp = (iou >= _TH) | selj
        curr[...] = jnp.where(supp, _NEG, cur)

        ok = m > (_NEG / 2.0)
        hit = oidx == t
        ox1[...] = jnp.where(hit, jnp.where(ok, bx1, 0.0), ox1[...])
        oy1[...] = jnp.where(hit, jnp.where(ok, by1, 0.0), oy1[...])
        ox2[...] = jnp.where(hit, jnp.where(ok, bx2, 0.0), ox2[...])
        oy2[...] = jnp.where(hit, jnp.where(ok, by2, 0.0), oy2[...])
        osc[...] = jnp.where(hit, jnp.where(ok, m, 0.0), osc[...])
        return carry

    lax.fori_loop(0, _POST, step, 0)


def _call(interpret=False):
    return pl.pallas_call(
        _nms_body,
        out_shape=[jax.ShapeDtypeStruct((8, _C), jnp.float32)] * 5,
        scratch_shapes=[pltpu.VMEM((_R, _C), jnp.float32)] * 6,
        interpret=interpret,
    )


def kernel(anchors, pred_deltas, scores):
    pad = _PAD - _N

    def col(x, i):
        return jnp.pad(x[:, i], (0, pad)).reshape(_R, _C)

    a = [col(anchors, i) for i in range(4)]
    d = [col(pred_deltas, i) for i in range(4)]
    s = jnp.pad(scores, (0, pad)).reshape(_R, _C)
    outs = _call()(*a, *d, s)
    cols = [o.reshape(-1)[:_POST] for o in outs]
    return jnp.stack(cols, axis=1)
